# baseline (speedup 1.0000x reference)
import jax
import jax.numpy as jnp
from jax.experimental import pallas as pl
from jax.experimental.pallas import tpu as pltpu

ROWS = 128
VOCAB = 100000
BROWS = 8
NBLK = ROWS // BROWS

_TINY = 1.1754943508222875e-38


def _rotl(x, d):
    return (x << jnp.uint32(d)) | (x >> jnp.uint32(32 - d))


def _threefry2x32(x0, x1):
    k0 = jnp.uint32(0)
    k1 = jnp.uint32(42)
    k2 = k0 ^ k1 ^ jnp.uint32(0x1BD11BDA)
    ks = (k0, k1, k2)
    rots = ((13, 15, 26, 6), (17, 29, 16, 24))
    x0 = x0 + ks[0]
    x1 = x1 + ks[1]
    for i in range(5):
        for d in rots[i % 2]:
            x0 = x0 + x1
            x1 = _rotl(x1, d)
            x1 = x0 ^ x1
        x0 = x0 + ks[(i + 1) % 3]
        x1 = x1 + ks[(i + 2) % 3] + jnp.uint32(i + 1)
    return x0, x1


def _bits_to_gumbel(bits):
    fb = (bits >> jnp.uint32(9)) | jnp.uint32(0x3F800000)
    floats = jax.lax.bitcast_convert_type(fb, jnp.float32) - jnp.float32(1.0)
    u = jnp.maximum(jnp.float32(_TINY), floats)
    return -jnp.log(-jnp.log(u))


def _mn_kernel(x_ref, action_ref, logp_ref):
    j = pl.program_id(0)
    x = x_ref[...]

    base = (j * (BROWS * VOCAB)).astype(jnp.uint32)
    col = jax.lax.broadcasted_iota(jnp.uint32, (BROWS, VOCAB), 1)
    row = jax.lax.broadcasted_iota(jnp.uint32, (BROWS, VOCAB), 0)
    cnt = row * jnp.uint32(VOCAB) + col + base
    o0, o1 = _threefry2x32(jnp.zeros_like(cnt), cnt)
    g = _bits_to_gumbel(o0 ^ o1)

    m = jnp.max(x, axis=1, keepdims=True)
    s = jnp.sum(jnp.exp(x - m), axis=1, keepdims=True)

    z = x + g
    cz = jnp.max(z, axis=1, keepdims=True)
    colr = jax.lax.broadcasted_iota(jnp.int32, (BROWS, VOCAB), 1)
    cidx = jnp.min(jnp.where(z == cz, colr, jnp.int32(VOCAB)), axis=1,
                   keepdims=True)
    xat = jnp.sum(jnp.where(colr == cidx, x, jnp.float32(0.0)), axis=1,
                  keepdims=True)

    action_ref[...] = cidx
    logp_ref[...] = xat - (m + jnp.log(s))


def kernel(features):
    action, logp = pl.pallas_call(
        _mn_kernel,
        grid=(NBLK,),
        in_specs=[pl.BlockSpec((BROWS, VOCAB), lambda j: (j, 0))],
        out_specs=[
            pl.BlockSpec((BROWS, 1), lambda j: (j, 0)),
            pl.BlockSpec((BROWS, 1), lambda j: (j, 0)),
        ],
        out_shape=[
            jax.ShapeDtypeStruct((ROWS, 1), jnp.int32),
            jax.ShapeDtypeStruct((ROWS, 1), jnp.float32),
        ],
        compiler_params=pltpu.CompilerParams(
            dimension_semantics=("arbitrary",),
        ),
    )(features)
    return action.reshape(ROWS), logp.reshape(ROWS)
